# baseline (speedup 1.0000x reference)
import jax
import jax.numpy as jnp
from jax import lax
from jax.experimental import pallas as pl
from jax.experimental.pallas import tpu as pltpu
from jax.experimental.pallas import tpu_sc as plsc

N = 5000
L = 16
NPAD = 5120
NCHUNK = NPAD // L
U = 4
USHIFT = 6
LPAD = NPAD + U * L
SCORE_THR = 0.1
NMS_THR = 0.01


def _nms_sc_body(x1h, y1h, x2h, y2h, sh, ordh,
                 ox1h, oy1h, ox2h, oy2h, osh,
                 x1, y1, x2, y2, s, ordv, keep, lst, lstb,
                 q1, q2, q3, q4, q5):
    cid = lax.axis_index("c")
    sid = lax.axis_index("s")
    lane = lax.broadcasted_iota(jnp.int32, (L,), 0)

    @pl.when(jnp.logical_and(cid == 0, sid == 0))
    def _():
        pltpu.sync_copy(x1h, x1)
        pltpu.sync_copy(y1h, y1)
        pltpu.sync_copy(x2h, x2)
        pltpu.sync_copy(y2h, y2)
        pltpu.sync_copy(sh, s)
        pltpu.sync_copy(ordh, ordv)

        def init_chunk(k, wv):
            sl = pl.ds(k * L, L)
            keep[sl] = jnp.zeros((L,), jnp.float32)
            ov = ordv[sl]
            sg = plsc.load_gather(s, [ov])
            m = jnp.logical_and(sg > SCORE_THR, (k * L + lane) < N)
            cs = jnp.cumsum(m.astype(jnp.int32))
            packed = lax.shift_left(k * L + lane, 13) + ov
            plsc.store_scatter(lst, [wv + cs - 1], packed, mask=m)
            return wv + plsc.all_reduce_population_count(m)

        wv0 = lax.fori_loop(0, NCHUNK, init_chunk,
                            jnp.zeros((L,), jnp.int32))

        def one_window(src, dst, n):
            c0 = src[pl.ds(0, L)]
            co = jnp.where(lane < n, jnp.bitwise_and(c0, 8191), 0)
            wx1 = plsc.load_gather(x1, [co])
            wy1 = plsc.load_gather(y1, [co])
            wx2 = plsc.load_gather(x2, [co])
            wy2 = plsc.load_gather(y2, [co])
            war = (wx2 - wx1 + 1.0) * (wy2 - wy1 + 1.0)

            kb = lane == 0
            for i in range(1, L):
                jvi = jnp.full((L,), co[i], jnp.int32)
                ix1 = plsc.load_gather(x1, [jvi])
                iy1 = plsc.load_gather(y1, [jvi])
                ix2 = plsc.load_gather(x2, [jvi])
                iy2 = plsc.load_gather(y2, [jvi])
                iar = (ix2 - ix1 + 1.0) * (iy2 - iy1 + 1.0)
                xx1 = jnp.maximum(ix1, wx1)
                yy1 = jnp.maximum(iy1, wy1)
                xx2 = jnp.minimum(ix2, wx2)
                yy2 = jnp.minimum(iy2, wy2)
                ww = jnp.maximum(0.0, xx2 - xx1 + 1.0)
                hh = jnp.maximum(0.0, yy2 - yy1 + 1.0)
                inter = ww * hh
                iou = inter / (iar + war - inter)
                conflict = jnp.logical_and(
                    kb, jnp.logical_and(lane < i, iou > NMS_THR))
                pc = plsc.all_reduce_population_count(conflict)
                ok = jnp.logical_and(lane == i,
                                     jnp.logical_and(pc == 0, i < n))
                kb = jnp.logical_or(kb, ok)

            jp = lax.shift_right_logical(c0, 13)
            plsc.store_scatter(keep, [jp], jnp.ones((L,), jnp.float32),
                               mask=kb)

            kbi = kb.astype(jnp.int32)
            kparams = []
            for l in range(L):
                vl = jnp.full((L,), kbi[l], jnp.int32) == 1
                jvl = jnp.full((L,), co[l], jnp.int32)
                lx1 = plsc.load_gather(x1, [jvl])
                ly1 = plsc.load_gather(y1, [jvl])
                lx2 = plsc.load_gather(x2, [jvl])
                ly2 = plsc.load_gather(y2, [jvl])
                lar = (lx2 - lx1 + 1.0) * (ly2 - ly1 + 1.0)
                kparams.append((vl, lx1, ly1, lx2, ly2, lar))

            nm = jnp.maximum(n - L, 0)
            ngroups = lax.shift_right_logical(nm + (U * L - 1), USHIFT)

            def sweep(g, wv):
                survs = []
                for u in range(U):
                    base = L + g * (U * L) + u * L
                    idx = src[pl.ds(base, L)]
                    inb = (base + lane) < n
                    idxc = jnp.where(inb, idx, 0)
                    cid = jnp.bitwise_and(idxc, 8191)
                    cx1 = plsc.load_gather(x1, [cid])
                    cy1 = plsc.load_gather(y1, [cid])
                    cx2 = plsc.load_gather(x2, [cid])
                    cy2 = plsc.load_gather(y2, [cid])
                    car = (cx2 - cx1 + 1.0) * (cy2 - cy1 + 1.0)
                    supp = jnp.zeros((L,), jnp.bool_)
                    for vl, lx1, ly1, lx2, ly2, lar in kparams:
                        xx1 = jnp.maximum(lx1, cx1)
                        yy1 = jnp.maximum(ly1, cy1)
                        xx2 = jnp.minimum(lx2, cx2)
                        yy2 = jnp.minimum(ly2, cy2)
                        ww = jnp.maximum(0.0, xx2 - xx1 + 1.0)
                        hh = jnp.maximum(0.0, yy2 - yy1 + 1.0)
                        inter = ww * hh
                        iou = inter / (lar + car - inter)
                        supp = jnp.logical_or(
                            supp, jnp.logical_and(vl, iou > NMS_THR))
                    surv = jnp.logical_and(inb, jnp.logical_not(supp))
                    cs = jnp.cumsum(surv.astype(jnp.int32))
                    survs.append((idxc, surv, cs))
                for idxc, surv, cs in survs:
                    plsc.store_scatter(dst, [wv + cs - 1], idxc, mask=surv)
                    wv = wv + plsc.all_reduce_population_count(surv)
                return wv

            wv = lax.fori_loop(0, ngroups, sweep,
                               jnp.zeros((L,), jnp.int32))
            return wv[0]

        def body(n):
            n1 = one_window(lst, lstb, n)
            return lax.cond(n1 > 0,
                            lambda: one_window(lstb, lst, n1),
                            lambda: n1)

        lax.while_loop(lambda n: n > 0, body, wv0[0])

        def out_chunk(k, _):
            sl = pl.ds(k * L, L)
            ov = ordv[sl]
            kf = keep[sl]
            q1[sl] = plsc.load_gather(x1, [ov]) * kf
            q2[sl] = plsc.load_gather(y1, [ov]) * kf
            q3[sl] = plsc.load_gather(x2, [ov]) * kf
            q4[sl] = plsc.load_gather(y2, [ov]) * kf
            q5[sl] = plsc.load_gather(s, [ov]) * kf
            return 0

        lax.fori_loop(0, NCHUNK, out_chunk, 0)
        pltpu.sync_copy(q1, ox1h)
        pltpu.sync_copy(q2, oy1h)
        pltpu.sync_copy(q3, ox2h)
        pltpu.sync_copy(q4, oy2h)
        pltpu.sync_copy(q5, osh)


_nms_sc = pl.kernel(
    _nms_sc_body,
    out_type=[jax.ShapeDtypeStruct((NPAD,), jnp.float32)] * 5,
    mesh=plsc.VectorSubcoreMesh(core_axis_name="c", subcore_axis_name="s",
                                num_cores=2, num_subcores=16),
    compiler_params=pltpu.CompilerParams(needs_layout_passes=False),
    scratch_types=[
        pltpu.VMEM((NPAD,), jnp.float32),
        pltpu.VMEM((NPAD,), jnp.float32),
        pltpu.VMEM((NPAD,), jnp.float32),
        pltpu.VMEM((NPAD,), jnp.float32),
        pltpu.VMEM((NPAD,), jnp.float32),
        pltpu.VMEM((NPAD,), jnp.int32),
        pltpu.VMEM((NPAD,), jnp.float32),
        pltpu.VMEM((LPAD,), jnp.int32),
        pltpu.VMEM((LPAD,), jnp.int32),
        pltpu.VMEM((NPAD,), jnp.float32),
        pltpu.VMEM((NPAD,), jnp.float32),
        pltpu.VMEM((NPAD,), jnp.float32),
        pltpu.VMEM((NPAD,), jnp.float32),
        pltpu.VMEM((NPAD,), jnp.float32),
    ],
)


@jax.jit
def kernel(boxes, scores):
    order = jnp.argsort(-scores).astype(jnp.int32)
    pad = NPAD - N
    x1 = jnp.pad(boxes[:, 0], (0, pad))
    y1 = jnp.pad(boxes[:, 1], (0, pad))
    x2 = jnp.pad(boxes[:, 2], (0, pad))
    y2 = jnp.pad(boxes[:, 3], (0, pad))
    sp = jnp.pad(scores, (0, pad))
    op = jnp.pad(order, (0, pad))
    res = _nms_sc(x1, y1, x2, y2, sp, op)
    return jnp.stack(res, axis=1)[:N]


# speedup vs baseline: 1.2397x; 1.2397x over previous
import jax
import jax.numpy as jnp
from jax import lax
from jax.experimental import pallas as pl
from jax.experimental.pallas import tpu as pltpu
from jax.experimental.pallas import tpu_sc as plsc

N = 5000
L = 16
NPAD = 5120
NCHUNK = NPAD // L
U = 2
USHIFT = 5
LPAD = NPAD + U * L
SCORE_THR = 0.1
NMS_THR = 0.01


def _nms_sc_body(x1h, y1h, x2h, y2h, sh, ordh,
                 ox1h, oy1h, ox2h, oy2h, osh,
                 x1, y1, x2, y2, s, ordv, keep, lst, lstb,
                 q1, q2, q3, q4, q5):
    cid = lax.axis_index("c")
    sid = lax.axis_index("s")
    lane = lax.broadcasted_iota(jnp.int32, (L,), 0)

    @pl.when(jnp.logical_and(cid == 0, sid == 0))
    def _():
        pltpu.sync_copy(x1h, x1)
        pltpu.sync_copy(y1h, y1)
        pltpu.sync_copy(x2h, x2)
        pltpu.sync_copy(y2h, y2)
        pltpu.sync_copy(sh, s)
        pltpu.sync_copy(ordh, ordv)

        def init_chunk(k, wv):
            sl = pl.ds(k * L, L)
            keep[sl] = jnp.zeros((L,), jnp.float32)
            ov = ordv[sl]
            sg = plsc.load_gather(s, [ov])
            m = jnp.logical_and(sg > SCORE_THR, (k * L + lane) < N)
            cs = jnp.cumsum(m.astype(jnp.int32))
            packed = lax.shift_left(k * L + lane, 13) + ov
            plsc.store_scatter(lst, [wv + cs - 1], packed, mask=m)
            return wv + plsc.all_reduce_population_count(m)

        wv0 = lax.fori_loop(0, NCHUNK, init_chunk,
                            jnp.zeros((L,), jnp.int32))

        def one_window(src, dst, n):
            c0 = src[pl.ds(0, L)]
            co = jnp.where(lane < n, jnp.bitwise_and(c0, 8191), 0)
            wx1 = plsc.load_gather(x1, [co])
            wy1 = plsc.load_gather(y1, [co])
            wx2 = plsc.load_gather(x2, [co])
            wy2 = plsc.load_gather(y2, [co])
            war = (wx2 - wx1 + 1.0) * (wy2 - wy1 + 1.0)

            kb = lane == 0
            for i in range(1, L):
                jvi = jnp.full((L,), co[i], jnp.int32)
                ix1 = plsc.load_gather(x1, [jvi])
                iy1 = plsc.load_gather(y1, [jvi])
                ix2 = plsc.load_gather(x2, [jvi])
                iy2 = plsc.load_gather(y2, [jvi])
                iar = (ix2 - ix1 + 1.0) * (iy2 - iy1 + 1.0)
                xx1 = jnp.maximum(ix1, wx1)
                yy1 = jnp.maximum(iy1, wy1)
                xx2 = jnp.minimum(ix2, wx2)
                yy2 = jnp.minimum(iy2, wy2)
                ww = jnp.maximum(0.0, xx2 - xx1 + 1.0)
                hh = jnp.maximum(0.0, yy2 - yy1 + 1.0)
                inter = ww * hh
                iou = inter / (iar + war - inter)
                conflict = jnp.logical_and(
                    kb, jnp.logical_and(lane < i, iou > NMS_THR))
                pc = plsc.all_reduce_population_count(conflict)
                ok = jnp.logical_and(lane == i,
                                     jnp.logical_and(pc == 0, i < n))
                kb = jnp.logical_or(kb, ok)

            jp = lax.shift_right_logical(c0, 13)
            plsc.store_scatter(keep, [jp], jnp.ones((L,), jnp.float32),
                               mask=kb)

            kbi = kb.astype(jnp.int32)
            kparams = []
            for l in range(L):
                vl = jnp.full((L,), kbi[l], jnp.int32) == 1
                jvl = jnp.full((L,), co[l], jnp.int32)
                lx1 = plsc.load_gather(x1, [jvl])
                ly1 = plsc.load_gather(y1, [jvl])
                lx2 = plsc.load_gather(x2, [jvl])
                ly2 = plsc.load_gather(y2, [jvl])
                lar = (lx2 - lx1 + 1.0) * (ly2 - ly1 + 1.0)
                kparams.append((vl, lx1, ly1, lx2, ly2, lar))

            nm = jnp.maximum(n - L, 0)
            ngroups = lax.shift_right_logical(nm + (U * L - 1), USHIFT)

            def sweep(g, wv):
                survs = []
                for u in range(U):
                    base = L + g * (U * L) + u * L
                    idx = src[pl.ds(base, L)]
                    inb = (base + lane) < n
                    idxc = jnp.where(inb, idx, 0)
                    cid = jnp.bitwise_and(idxc, 8191)
                    cx1 = plsc.load_gather(x1, [cid])
                    cy1 = plsc.load_gather(y1, [cid])
                    cx2 = plsc.load_gather(x2, [cid])
                    cy2 = plsc.load_gather(y2, [cid])
                    car = (cx2 - cx1 + 1.0) * (cy2 - cy1 + 1.0)
                    supp = jnp.zeros((L,), jnp.bool_)
                    for vl, lx1, ly1, lx2, ly2, lar in kparams:
                        xx1 = jnp.maximum(lx1, cx1)
                        yy1 = jnp.maximum(ly1, cy1)
                        xx2 = jnp.minimum(lx2, cx2)
                        yy2 = jnp.minimum(ly2, cy2)
                        ww = jnp.maximum(0.0, xx2 - xx1 + 1.0)
                        hh = jnp.maximum(0.0, yy2 - yy1 + 1.0)
                        inter = ww * hh
                        iou = inter / (lar + car - inter)
                        supp = jnp.logical_or(
                            supp, jnp.logical_and(vl, iou > NMS_THR))
                    surv = jnp.logical_and(inb, jnp.logical_not(supp))
                    cs = jnp.cumsum(surv.astype(jnp.int32))
                    survs.append((idxc, surv, cs))
                for idxc, surv, cs in survs:
                    plsc.store_scatter(dst, [wv + cs - 1], idxc, mask=surv)
                    wv = wv + plsc.all_reduce_population_count(surv)
                return wv

            wv = lax.fori_loop(0, ngroups, sweep,
                               jnp.zeros((L,), jnp.int32))
            return wv[0]

        def body(n):
            n1 = one_window(lst, lstb, n)
            return lax.cond(n1 > 0,
                            lambda: one_window(lstb, lst, n1),
                            lambda: n1)

        lax.while_loop(lambda n: n > 0, body, wv0[0])

        def out_chunk(k, _):
            sl = pl.ds(k * L, L)
            ov = ordv[sl]
            kf = keep[sl]
            q1[sl] = plsc.load_gather(x1, [ov]) * kf
            q2[sl] = plsc.load_gather(y1, [ov]) * kf
            q3[sl] = plsc.load_gather(x2, [ov]) * kf
            q4[sl] = plsc.load_gather(y2, [ov]) * kf
            q5[sl] = plsc.load_gather(s, [ov]) * kf
            return 0

        lax.fori_loop(0, NCHUNK, out_chunk, 0)
        pltpu.sync_copy(q1, ox1h)
        pltpu.sync_copy(q2, oy1h)
        pltpu.sync_copy(q3, ox2h)
        pltpu.sync_copy(q4, oy2h)
        pltpu.sync_copy(q5, osh)


_nms_sc = pl.kernel(
    _nms_sc_body,
    out_type=[jax.ShapeDtypeStruct((NPAD,), jnp.float32)] * 5,
    mesh=plsc.VectorSubcoreMesh(core_axis_name="c", subcore_axis_name="s",
                                num_cores=2, num_subcores=16),
    compiler_params=pltpu.CompilerParams(needs_layout_passes=False),
    scratch_types=[
        pltpu.VMEM((NPAD,), jnp.float32),
        pltpu.VMEM((NPAD,), jnp.float32),
        pltpu.VMEM((NPAD,), jnp.float32),
        pltpu.VMEM((NPAD,), jnp.float32),
        pltpu.VMEM((NPAD,), jnp.float32),
        pltpu.VMEM((NPAD,), jnp.int32),
        pltpu.VMEM((NPAD,), jnp.float32),
        pltpu.VMEM((LPAD,), jnp.int32),
        pltpu.VMEM((LPAD,), jnp.int32),
        pltpu.VMEM((NPAD,), jnp.float32),
        pltpu.VMEM((NPAD,), jnp.float32),
        pltpu.VMEM((NPAD,), jnp.float32),
        pltpu.VMEM((NPAD,), jnp.float32),
        pltpu.VMEM((NPAD,), jnp.float32),
    ],
)


@jax.jit
def kernel(boxes, scores):
    order = jnp.argsort(-scores).astype(jnp.int32)
    pad = NPAD - N
    x1 = jnp.pad(boxes[:, 0], (0, pad))
    y1 = jnp.pad(boxes[:, 1], (0, pad))
    x2 = jnp.pad(boxes[:, 2], (0, pad))
    y2 = jnp.pad(boxes[:, 3], (0, pad))
    sp = jnp.pad(scores, (0, pad))
    op = jnp.pad(order, (0, pad))
    res = _nms_sc(x1, y1, x2, y2, sp, op)
    return jnp.stack(res, axis=1)[:N]
